# Initial kernel scaffold; baseline (speedup 1.0000x reference)
import jax
import jax.numpy as jnp
from jax.experimental import pallas as pl


def kernel(x):
    raise NotImplementedError("write your pallas kernel here")


# speedup vs baseline: 2.6218x; 2.6218x over previous
import functools

import jax
import jax.numpy as jnp
from jax.experimental import pallas as pl

N = 4096
BLOCK_ROWS = 512
CHUNK = 512


def _cumsum_block_kernel(x_ref, o_ref, *, chunk):
    x = x_ref[...]
    rows, n = x.shape
    nchunks = n // chunk
    col = jax.lax.broadcasted_iota(jnp.int32, (chunk, chunk), 1)
    row = jax.lax.broadcasted_iota(jnp.int32, (chunk, chunk), 0)
    tri = (row <= col).astype(jnp.float32)
    carry = jnp.zeros((rows, 1), jnp.float32)
    for c in range(nchunks):
        seg = x[:, c * chunk:(c + 1) * chunk]
        cs = jax.lax.dot(
            seg, tri,
            precision=jax.lax.Precision.HIGHEST,
            preferred_element_type=jnp.float32,
        ) + carry
        o_ref[:, c * chunk:(c + 1) * chunk] = cs
        carry = cs[:, chunk - 1:chunk]


def kernel(x):
    rows, n = x.shape
    grid = (rows // BLOCK_ROWS,)
    return pl.pallas_call(
        functools.partial(_cumsum_block_kernel, chunk=CHUNK),
        grid=grid,
        in_specs=[pl.BlockSpec((BLOCK_ROWS, n), lambda i: (i, 0))],
        out_specs=pl.BlockSpec((BLOCK_ROWS, n), lambda i: (i, 0)),
        out_shape=jax.ShapeDtypeStruct((rows, n), jnp.float32),
    )(x)


# speedup vs baseline: 5.9782x; 2.2802x over previous
import functools

import jax
import jax.numpy as jnp
from jax.experimental import pallas as pl

N = 4096
BLOCK_ROWS = 512
CHUNK = 512


def _cumsum_block_kernel(x_ref, o_ref, *, chunk):
    x = x_ref[...]
    rows, n = x.shape
    nchunks = n // chunk
    col = jax.lax.broadcasted_iota(jnp.int32, (chunk, chunk), 1)
    row = jax.lax.broadcasted_iota(jnp.int32, (chunk, chunk), 0)
    tri = (row <= col).astype(jnp.bfloat16)
    hi = x.astype(jnp.bfloat16)
    lo = (x - hi.astype(jnp.float32)).astype(jnp.bfloat16)
    carry = jnp.zeros((rows, 1), jnp.float32)
    for c in range(nchunks):
        sl = pl.ds(c * chunk, chunk)
        cs = (
            jax.lax.dot(hi[:, c * chunk:(c + 1) * chunk], tri,
                        preferred_element_type=jnp.float32)
            + jax.lax.dot(lo[:, c * chunk:(c + 1) * chunk], tri,
                          preferred_element_type=jnp.float32)
            + carry
        )
        o_ref[:, sl] = cs
        carry = cs[:, chunk - 1:chunk]


def kernel(x):
    rows, n = x.shape
    grid = (rows // BLOCK_ROWS,)
    return pl.pallas_call(
        functools.partial(_cumsum_block_kernel, chunk=CHUNK),
        grid=grid,
        in_specs=[pl.BlockSpec((BLOCK_ROWS, n), lambda i: (i, 0))],
        out_specs=pl.BlockSpec((BLOCK_ROWS, n), lambda i: (i, 0)),
        out_shape=jax.ShapeDtypeStruct((rows, n), jnp.float32),
    )(x)


# speedup vs baseline: 6.6477x; 1.1120x over previous
import functools

import jax
import jax.numpy as jnp
from jax.experimental import pallas as pl

N = 4096
BLOCK_ROWS = 512
CHUNK = 256


def _cumsum_block_kernel(x_ref, o_ref, *, chunk):
    x = x_ref[...]
    rows, n = x.shape
    nchunks = n // chunk
    col = jax.lax.broadcasted_iota(jnp.int32, (chunk, chunk), 1)
    row = jax.lax.broadcasted_iota(jnp.int32, (chunk, chunk), 0)
    tri = (row <= col).astype(jnp.bfloat16)
    hi = x.astype(jnp.bfloat16)
    lo = (x - hi.astype(jnp.float32)).astype(jnp.bfloat16)
    carry = jnp.zeros((rows, 1), jnp.float32)
    for c in range(nchunks):
        sl = pl.ds(c * chunk, chunk)
        cs = (
            jax.lax.dot(hi[:, c * chunk:(c + 1) * chunk], tri,
                        preferred_element_type=jnp.float32)
            + jax.lax.dot(lo[:, c * chunk:(c + 1) * chunk], tri,
                          preferred_element_type=jnp.float32)
            + carry
        )
        o_ref[:, sl] = cs
        carry = cs[:, chunk - 1:chunk]


def kernel(x):
    rows, n = x.shape
    grid = (rows // BLOCK_ROWS,)
    return pl.pallas_call(
        functools.partial(_cumsum_block_kernel, chunk=CHUNK),
        grid=grid,
        in_specs=[pl.BlockSpec((BLOCK_ROWS, n), lambda i: (i, 0))],
        out_specs=pl.BlockSpec((BLOCK_ROWS, n), lambda i: (i, 0)),
        out_shape=jax.ShapeDtypeStruct((rows, n), jnp.float32),
    )(x)
